# Initial kernel scaffold; baseline (speedup 1.0000x reference)
import jax
import jax.numpy as jnp
from jax.experimental import pallas as pl


def kernel(x, centers, sigmas, out_centers, input_rules, output_rules):
    raise NotImplementedError("write your pallas kernel here")


# speedup vs baseline: 2.0908x; 2.0908x over previous
import functools

import jax
import jax.numpy as jnp
from jax.experimental import pallas as pl
from jax.experimental.pallas import tpu as pltpu

B = 512
NVAR = 6
M = 7
R = 16384
OUT_M = 9
MP = 8
FCOLS = NVAR * MP
NQ = 32
RT = 2048


def _min_weights(xe_ref, cvec_ref, svec_ref, rules_ref):
    xe = xe_ref[...]
    c = cvec_ref[0:1, :]
    s = svec_ref[0:1, :]
    d = xe - c
    fuzz = jnp.exp(-(d * d) / (2.0 * s * s))

    rules = rules_ref[...]
    w = None
    for v in range(NVAR):
        idx = rules[v:v + 1, :] - (M * v)
        oh = (jax.lax.broadcasted_iota(jnp.int32, (MP, RT), 0) == idx
              ).astype(jnp.float32)
        fv = fuzz[:, MP * v:MP * (v + 1)]
        wv = jnp.dot(fv, oh, preferred_element_type=jnp.float32,
                     precision=jax.lax.Precision.HIGHEST)
        w = wv if w is None else jnp.minimum(w, wv)
    return w


def _anfis_body(xe_ref, cvec_ref, svec_ref, rules_ref, orules_ref, ocp_ref,
                out_ref, acc_ref, norm_ref):
    p = pl.program_id(0)
    i = pl.program_id(1)

    @pl.when((p == 0) & (i == 0))
    def _init():
        acc_ref[...] = jnp.zeros_like(acc_ref)
        norm_ref[...] = jnp.zeros_like(norm_ref)

    w = _min_weights(xe_ref, cvec_ref, svec_ref, rules_ref)

    @pl.when(p == 0)
    def _norm_pass():
        norm_ref[:, 0:1] += jnp.sum(w, axis=1, keepdims=True)

    @pl.when(p == 1)
    def _contract_pass():
        norm = jnp.maximum(norm_ref[:, 0:1], 1e-12)
        nw = (w / norm).astype(jnp.bfloat16).astype(jnp.float32)
        orules = orules_ref[...]
        or0 = orules[:, 0:1]
        or1 = orules[:, 1:2]
        jq = jax.lax.broadcasted_iota(jnp.int32, (RT, NQ), 1)
        q = ((jq == or0) | (jq == or1)).astype(jnp.float32)
        acc_ref[...] += jnp.dot(nw, q, preferred_element_type=jnp.float32,
                                precision=jax.lax.Precision.HIGHEST)

    @pl.when((p == 1) & (i == pl.num_programs(1) - 1))
    def _fini():
        acc = acc_ref[...]
        ocb = ocp_ref[...].astype(jnp.bfloat16).astype(jnp.float32)
        res = jnp.dot(acc, ocb, preferred_element_type=jnp.float32,
                      precision=jax.lax.Precision.HIGHEST)
        out_ref[...] = res[:, 0:2]


@jax.jit
def kernel(x, centers, sigmas, out_centers, input_rules, output_rules):
    xe = jnp.repeat(x, MP, axis=1)
    cpad = jnp.pad(centers, ((0, 0), (0, MP - M)))
    spad = jnp.pad(sigmas, ((0, 0), (0, MP - M)),
                   constant_values=1.0)
    cvec = jnp.tile(cpad.reshape(1, FCOLS), (8, 1))
    svec = jnp.tile(spad.reshape(1, FCOLS), (8, 1))
    rules_t = jnp.pad(input_rules.T, ((0, 2), (0, 0)))
    orules_p = jnp.pad(output_rules, ((0, 0), (0, 6)))
    ocp = jnp.zeros((NQ, 8), jnp.float32)
    ocp = ocp.at[0:OUT_M, 0].set(out_centers[0:OUT_M])
    ocp = ocp.at[OUT_M:2 * OUT_M, 1].set(out_centers[OUT_M:2 * OUT_M])

    grid = (2, R // RT)
    out = pl.pallas_call(
        _anfis_body,
        grid=grid,
        in_specs=[
            pl.BlockSpec((B, FCOLS), lambda p, i: (0, 0)),
            pl.BlockSpec((8, FCOLS), lambda p, i: (0, 0)),
            pl.BlockSpec((8, FCOLS), lambda p, i: (0, 0)),
            pl.BlockSpec((8, RT), lambda p, i: (0, i)),
            pl.BlockSpec((RT, 8), lambda p, i: (i, 0)),
            pl.BlockSpec((NQ, 8), lambda p, i: (0, 0)),
        ],
        out_specs=pl.BlockSpec((B, 2), lambda p, i: (0, 0)),
        out_shape=jax.ShapeDtypeStruct((B, 2), jnp.float32),
        scratch_shapes=[pltpu.VMEM((B, NQ), jnp.float32),
                        pltpu.VMEM((B, 8), jnp.float32)],
        compiler_params=pltpu.CompilerParams(
            dimension_semantics=("arbitrary", "arbitrary"),
        ),
    )(xe, cvec, svec, rules_t, orules_p, ocp)
    return out


# speedup vs baseline: 2.4220x; 1.1584x over previous
import functools

import jax
import jax.numpy as jnp
from jax import lax
from jax.experimental import pallas as pl
from jax.experimental.pallas import tpu as pltpu
from jax.experimental.pallas import tpu_sc as plsc

B = 512
NVAR = 6
M = 7
R = 16384
OUT_M = 9
NC = 2
NS = 16
NW = NC * NS
L = 16
NPAIR = 3
PW = M * M
CH = 2048
NCH = R // CH
NOUT2 = 2 * OUT_M


def _bf16_round(v):
    y = lax.bitcast_convert_type(v, jnp.int32)
    odd = lax.shift_right_logical(y, 16) & jnp.int32(1)
    r = (y + jnp.int32(0x7FFF) + odd) & jnp.int32(-65536)
    return lax.bitcast_convert_type(r, jnp.float32)


def _sc_body(xw_hbm, csp_hbm, rules_hbm, oc_hbm, out_hbm,
             xv_ref, csv_ref, oc_ref, pt0_ref, pt1_ref, pt2_ref,
             rbuf_ref, code_ref, obase_ref, s_ref, outb_ref):
    wid = lax.axis_index("s") * NC + lax.axis_index("c")
    iota = lax.iota(jnp.int32, L)

    pltpu.sync_copy(xw_hbm.at[pl.ds(wid * (NVAR * L), NVAR * L)], xv_ref)
    pltpu.sync_copy(csp_hbm, csv_ref)
    pltpu.sync_copy(oc_hbm, oc_ref)

    ocr0 = _bf16_round(oc_ref[pl.ds(0, L)])
    ocr1 = _bf16_round(oc_ref[pl.ds(L, L)])

    pts = (pt0_ref, pt1_ref, pt2_ref)
    for a in range(NPAIR):
        va, vb = 2 * a, 2 * a + 1
        xa = xv_ref[pl.ds(va * L, L)]
        xb = xv_ref[pl.ds(vb * L, L)]
        ca_row = csv_ref[pl.ds(va * L, L)]
        cb_row = csv_ref[pl.ds(vb * L, L)]
        sa_row = csv_ref[pl.ds((NVAR + va) * L, L)]
        sb_row = csv_ref[pl.ds((NVAR + vb) * L, L)]
        fa, fb = [], []
        for m in range(M):
            da = xa - ca_row[m]
            ka = 0.5 / (jnp.full((L,), sa_row[m]) * jnp.full((L,), sa_row[m]))
            fa.append(jnp.exp(-(da * da) * ka))
            db = xb - cb_row[m]
            kb = 0.5 / (jnp.full((L,), sb_row[m]) * jnp.full((L,), sb_row[m]))
            fb.append(jnp.exp(-(db * db) * kb))
        for i in range(M):
            for j in range(M):
                pts[a][pl.ds((M * i + j) * L, L)] = jnp.minimum(fa[i], fb[j])

    for jj in range(NOUT2):
        s_ref[pl.ds(jj * L, L)] = jnp.zeros((L,), jnp.float32)

    def _load_chunk(ch):
        pltpu.sync_copy(rules_hbm.at[pl.ds(ch * 8 * CH, 8 * CH)], rbuf_ref)

        def _codes(i, carry):
            def fld(f):
                return rbuf_ref[pl.ds(f * CH + i * L, L)]
            code_ref[pl.ds(0 * CH + i * L, L)] = (
                fld(0) * M + (fld(1) - M)) * L
            code_ref[pl.ds(1 * CH + i * L, L)] = (
                (fld(2) - 2 * M) * M + (fld(3) - 3 * M)) * L
            code_ref[pl.ds(2 * CH + i * L, L)] = (
                (fld(4) - 4 * M) * M + (fld(5) - 5 * M)) * L
            obase_ref[pl.ds(0 * CH + i * L, L)] = fld(6) * L
            obase_ref[pl.ds(1 * CH + i * L, L)] = fld(7) * L
            return carry

        lax.fori_loop(0, CH // L, _codes, 0, unroll=False)

    def _gather_w(c0v, c1v, c2v, t):
        w0 = plsc.load_gather(pt0_ref, [iota + c0v[t]])
        w1 = plsc.load_gather(pt1_ref, [iota + c1v[t]])
        w2 = plsc.load_gather(pt2_ref, [iota + c2v[t]])
        return jnp.minimum(jnp.minimum(w0, w1), w2)

    def _norm_block(k, acc):
        c0v = code_ref[pl.ds(0 * CH + k * L, L)]
        c1v = code_ref[pl.ds(1 * CH + k * L, L)]
        c2v = code_ref[pl.ds(2 * CH + k * L, L)]
        for t in range(L):
            acc = acc + _gather_w(c0v, c1v, c2v, t)
        return acc

    norm = jnp.zeros((L,), jnp.float32)
    for ch in range(NCH):
        _load_chunk(ch)
        norm = lax.fori_loop(0, CH // L, _norm_block, norm, unroll=False)

    inv = 1.0 / jnp.maximum(norm, 1e-12)

    def _accum_block(k, carry):
        c0v = code_ref[pl.ds(0 * CH + k * L, L)]
        c1v = code_ref[pl.ds(1 * CH + k * L, L)]
        c2v = code_ref[pl.ds(2 * CH + k * L, L)]
        o0v = obase_ref[pl.ds(0 * CH + k * L, L)]
        o1v = obase_ref[pl.ds(1 * CH + k * L, L)]
        for t in range(L):
            nw = _bf16_round(_gather_w(c0v, c1v, c2v, t) * inv)
            plsc.addupdate_scatter(s_ref, [iota + o0v[t]], nw)
            plsc.addupdate_scatter(s_ref, [iota + o1v[t]], nw)
        return carry

    for ch in range(NCH):
        _load_chunk(ch)
        lax.fori_loop(0, CH // L, _accum_block, 0, unroll=False)

    acc0 = jnp.zeros((L,), jnp.float32)
    for jj in range(OUT_M):
        acc0 = acc0 + s_ref[pl.ds(jj * L, L)] * ocr0[jj]
    acc1 = jnp.zeros((L,), jnp.float32)
    for jj in range(OUT_M, NOUT2):
        scal = ocr0[jj] if jj < L else ocr1[jj - L]
        acc1 = acc1 + s_ref[pl.ds(jj * L, L)] * scal
    outb_ref[pl.ds(0, L)] = acc0
    outb_ref[pl.ds(L, L)] = acc1
    pltpu.sync_copy(outb_ref, out_hbm.at[pl.ds(wid * 2 * L, 2 * L)])


@jax.jit
def kernel(x, centers, sigmas, out_centers, input_rules, output_rules):
    xw = x.T.reshape(NVAR, NW, L).transpose(1, 0, 2).reshape(-1)
    csp = jnp.pad(jnp.concatenate([centers, sigmas], axis=0),
                  ((0, 0), (0, L - M)),
                  constant_values=1.0).reshape(-1)
    rules8 = jnp.concatenate([input_rules.T, output_rules.T],
                             axis=0).astype(jnp.int32)
    rulesf = rules8.reshape(8, NCH, CH).transpose(1, 0, 2).reshape(-1)
    ocp = jnp.pad(out_centers, (0, 32 - NOUT2))

    mesh = plsc.VectorSubcoreMesh(core_axis_name="c", subcore_axis_name="s",
                                  num_cores=NC, num_subcores=NS)
    run = functools.partial(
        pl.kernel,
        out_type=jax.ShapeDtypeStruct((NW * 2 * L,), jnp.float32),
        mesh=mesh,
        scratch_types=[
            pltpu.VMEM((NVAR * L,), jnp.float32),
            pltpu.VMEM((2 * NVAR * L,), jnp.float32),
            pltpu.VMEM((32,), jnp.float32),
            pltpu.VMEM((PW * L + L,), jnp.float32),
            pltpu.VMEM((PW * L + L,), jnp.float32),
            pltpu.VMEM((PW * L + L,), jnp.float32),
            pltpu.VMEM((8 * CH,), jnp.int32),
            pltpu.VMEM((NPAIR * CH,), jnp.int32),
            pltpu.VMEM((2 * CH,), jnp.int32),
            pltpu.VMEM((NOUT2 * L,), jnp.float32),
            pltpu.VMEM((2 * L,), jnp.float32),
        ],
        compiler_params=pltpu.CompilerParams(needs_layout_passes=False),
    )(_sc_body)
    flat = run(xw, csp, rulesf, ocp)
    return flat.reshape(NW, 2, L).transpose(0, 2, 1).reshape(B, 2)


# speedup vs baseline: 3.1118x; 1.2848x over previous
import functools

import jax
import jax.numpy as jnp
from jax import lax
from jax.experimental import pallas as pl
from jax.experimental.pallas import tpu as pltpu
from jax.experimental.pallas import tpu_sc as plsc

B = 512
NVAR = 6
M = 7
R = 16384
OUT_M = 9
NC = 2
NS = 16
NW = NC * NS
L = 16
NPAIR = 3
PW = M * M
CH = 2048
NCH = R // CH
NOUT2 = 2 * OUT_M


def _bf16_round(v):
    y = lax.bitcast_convert_type(v, jnp.int32)
    odd = lax.shift_right_logical(y, 16) & jnp.int32(1)
    r = (y + jnp.int32(0x7FFF) + odd) & jnp.int32(-65536)
    return lax.bitcast_convert_type(r, jnp.float32)


def _sc_body(xw_hbm, csp_hbm, rules_hbm, oc_hbm, out_hbm,
             xv_ref, csv_ref, oc_ref, pt0_ref, pt1_ref, pt2_ref,
             rbuf_ref, code_ref, obase_ref, s_ref, outb_ref):
    wid = lax.axis_index("s") * NC + lax.axis_index("c")
    iota = lax.iota(jnp.int32, L)

    pltpu.sync_copy(xw_hbm.at[pl.ds(wid * (NVAR * L), NVAR * L)], xv_ref)
    pltpu.sync_copy(csp_hbm, csv_ref)
    pltpu.sync_copy(oc_hbm, oc_ref)

    ocr0 = _bf16_round(oc_ref[pl.ds(0, L)])
    ocr1 = _bf16_round(oc_ref[pl.ds(L, L)])

    pts = (pt0_ref, pt1_ref, pt2_ref)
    for a in range(NPAIR):
        va, vb = 2 * a, 2 * a + 1
        xa = xv_ref[pl.ds(va * L, L)]
        xb = xv_ref[pl.ds(vb * L, L)]
        ca_row = csv_ref[pl.ds(va * L, L)]
        cb_row = csv_ref[pl.ds(vb * L, L)]
        sa_row = csv_ref[pl.ds((NVAR + va) * L, L)]
        sb_row = csv_ref[pl.ds((NVAR + vb) * L, L)]
        fa, fb = [], []
        for m in range(M):
            da = xa - ca_row[m]
            ka = 0.5 / (jnp.full((L,), sa_row[m]) * jnp.full((L,), sa_row[m]))
            fa.append(jnp.exp(-(da * da) * ka))
            db = xb - cb_row[m]
            kb = 0.5 / (jnp.full((L,), sb_row[m]) * jnp.full((L,), sb_row[m]))
            fb.append(jnp.exp(-(db * db) * kb))
        for i in range(M):
            for j in range(M):
                pts[a][pl.ds((M * i + j) * L, L)] = jnp.minimum(fa[i], fb[j])

    for jj in range(NOUT2):
        s_ref[pl.ds(jj * L, L)] = jnp.zeros((L,), jnp.float32)

    for ch in range(NCH):
        pltpu.sync_copy(rules_hbm.at[pl.ds(ch * 8 * CH, 8 * CH)], rbuf_ref)

        def _codes(i, carry, ch=ch):
            def fld(f):
                return rbuf_ref[pl.ds(f * CH + i * L, L)]
            g = ch * CH + i * L
            code_ref[pl.ds(0 * R + g, L)] = (
                fld(0) * M + (fld(1) - M)) * L
            code_ref[pl.ds(1 * R + g, L)] = (
                (fld(2) - 2 * M) * M + (fld(3) - 3 * M)) * L
            code_ref[pl.ds(2 * R + g, L)] = (
                (fld(4) - 4 * M) * M + (fld(5) - 5 * M)) * L
            obase_ref[pl.ds(0 * R + g, L)] = fld(6) * L
            obase_ref[pl.ds(1 * R + g, L)] = fld(7) * L
            return carry

        lax.fori_loop(0, CH // L, _codes, 0, unroll=False)

    def _gather_w(c0v, c1v, c2v, t):
        w0 = pt0_ref[pl.ds(c0v[t], L)]
        w1 = pt1_ref[pl.ds(c1v[t], L)]
        w2 = pt2_ref[pl.ds(c2v[t], L)]
        return jnp.minimum(jnp.minimum(w0, w1), w2)

    def _norm_block(k, accs):
        c0v = code_ref[pl.ds(0 * R + k * L, L)]
        c1v = code_ref[pl.ds(1 * R + k * L, L)]
        c2v = code_ref[pl.ds(2 * R + k * L, L)]
        accs = list(accs)
        for t in range(L):
            accs[t % 4] = accs[t % 4] + _gather_w(c0v, c1v, c2v, t)
        return tuple(accs)

    z = jnp.zeros((L,), jnp.float32)
    a0, a1, a2, a3 = lax.fori_loop(0, R // L, _norm_block, (z, z, z, z),
                                   unroll=False)
    norm = (a0 + a1) + (a2 + a3)

    inv = 1.0 / jnp.maximum(norm, 1e-12)

    for c in range(PW):
        pt0_ref[pl.ds(c * L, L)] = pt0_ref[pl.ds(c * L, L)] * inv
        pt1_ref[pl.ds(c * L, L)] = pt1_ref[pl.ds(c * L, L)] * inv
        pt2_ref[pl.ds(c * L, L)] = pt2_ref[pl.ds(c * L, L)] * inv

    def _accum_block(k, carry):
        c0v = code_ref[pl.ds(0 * R + k * L, L)]
        c1v = code_ref[pl.ds(1 * R + k * L, L)]
        c2v = code_ref[pl.ds(2 * R + k * L, L)]
        o0v = obase_ref[pl.ds(0 * R + k * L, L)]
        o1v = obase_ref[pl.ds(1 * R + k * L, L)]
        for t in range(L):
            nw = _bf16_round(_gather_w(c0v, c1v, c2v, t))
            plsc.addupdate_scatter(s_ref, [iota + o0v[t]], nw)
            plsc.addupdate_scatter(s_ref, [iota + o1v[t]], nw)
        return carry

    lax.fori_loop(0, R // L, _accum_block, 0, unroll=False)

    acc0 = jnp.zeros((L,), jnp.float32)
    for jj in range(OUT_M):
        acc0 = acc0 + s_ref[pl.ds(jj * L, L)] * ocr0[jj]
    acc1 = jnp.zeros((L,), jnp.float32)
    for jj in range(OUT_M, NOUT2):
        scal = ocr0[jj] if jj < L else ocr1[jj - L]
        acc1 = acc1 + s_ref[pl.ds(jj * L, L)] * scal
    outb_ref[pl.ds(0, L)] = acc0
    outb_ref[pl.ds(L, L)] = acc1
    pltpu.sync_copy(outb_ref, out_hbm.at[pl.ds(wid * 2 * L, 2 * L)])


@jax.jit
def kernel(x, centers, sigmas, out_centers, input_rules, output_rules):
    xw = x.T.reshape(NVAR, NW, L).transpose(1, 0, 2).reshape(-1)
    csp = jnp.pad(jnp.concatenate([centers, sigmas], axis=0),
                  ((0, 0), (0, L - M)),
                  constant_values=1.0).reshape(-1)
    rules8 = jnp.concatenate([input_rules.T, output_rules.T],
                             axis=0).astype(jnp.int32)
    rulesf = rules8.reshape(8, NCH, CH).transpose(1, 0, 2).reshape(-1)
    ocp = jnp.pad(out_centers, (0, 32 - NOUT2))

    mesh = plsc.VectorSubcoreMesh(core_axis_name="c", subcore_axis_name="s",
                                  num_cores=NC, num_subcores=NS)
    run = functools.partial(
        pl.kernel,
        out_type=jax.ShapeDtypeStruct((NW * 2 * L,), jnp.float32),
        mesh=mesh,
        scratch_types=[
            pltpu.VMEM((NVAR * L,), jnp.float32),
            pltpu.VMEM((2 * NVAR * L,), jnp.float32),
            pltpu.VMEM((32,), jnp.float32),
            pltpu.VMEM((PW * L + L,), jnp.float32),
            pltpu.VMEM((PW * L + L,), jnp.float32),
            pltpu.VMEM((PW * L + L,), jnp.float32),
            pltpu.VMEM((8 * CH,), jnp.int32),
            pltpu.VMEM((NPAIR * R,), jnp.int32),
            pltpu.VMEM((2 * R,), jnp.int32),
            pltpu.VMEM((NOUT2 * L,), jnp.float32),
            pltpu.VMEM((2 * L,), jnp.float32),
        ],
        compiler_params=pltpu.CompilerParams(needs_layout_passes=False),
    )(_sc_body)
    flat = run(xw, csp, rulesf, ocp)
    return flat.reshape(NW, 2, L).transpose(0, 2, 1).reshape(B, 2)


# speedup vs baseline: 3.4306x; 1.1025x over previous
import functools

import jax
import jax.numpy as jnp
from jax import lax
from jax.experimental import pallas as pl
from jax.experimental.pallas import tpu as pltpu
from jax.experimental.pallas import tpu_sc as plsc

B = 512
NVAR = 6
M = 7
R = 16384
OUT_M = 9
NC = 2
NS = 16
NW = NC * NS
L = 16
NPAIR = 3
PW = M * M
CH = 2048
NCH = R // CH
NOUT2 = 2 * OUT_M


def _bf16_round(v):
    y = lax.bitcast_convert_type(v, jnp.int32)
    odd = lax.shift_right_logical(y, 16) & jnp.int32(1)
    r = (y + jnp.int32(0x7FFF) + odd) & jnp.int32(-65536)
    return lax.bitcast_convert_type(r, jnp.float32)


def _sc_body(xw_hbm, csp_hbm, rules_hbm, oc_hbm, out_hbm,
             xv_ref, csv_ref, oc_ref, pt0_ref, pt1_ref, pt2_ref,
             rbuf_ref, code_ref, obase_ref, s_ref, outb_ref):
    wid = lax.axis_index("s") * NC + lax.axis_index("c")
    iota = lax.iota(jnp.int32, L)

    pltpu.sync_copy(xw_hbm.at[pl.ds(wid * (NVAR * L), NVAR * L)], xv_ref)
    pltpu.sync_copy(csp_hbm, csv_ref)
    pltpu.sync_copy(oc_hbm, oc_ref)

    ocr0 = _bf16_round(oc_ref[pl.ds(0, L)])
    ocr1 = _bf16_round(oc_ref[pl.ds(L, L)])

    pts = (pt0_ref, pt1_ref, pt2_ref)
    for a in range(NPAIR):
        va, vb = 2 * a, 2 * a + 1
        xa = xv_ref[pl.ds(va * L, L)]
        xb = xv_ref[pl.ds(vb * L, L)]
        ca_row = csv_ref[pl.ds(va * L, L)]
        cb_row = csv_ref[pl.ds(vb * L, L)]
        sa_row = csv_ref[pl.ds((NVAR + va) * L, L)]
        sb_row = csv_ref[pl.ds((NVAR + vb) * L, L)]
        fa, fb = [], []
        for m in range(M):
            da = xa - ca_row[m]
            ka = 0.5 / (jnp.full((L,), sa_row[m]) * jnp.full((L,), sa_row[m]))
            fa.append(jnp.exp(-(da * da) * ka))
            db = xb - cb_row[m]
            kb = 0.5 / (jnp.full((L,), sb_row[m]) * jnp.full((L,), sb_row[m]))
            fb.append(jnp.exp(-(db * db) * kb))
        for i in range(M):
            for j in range(M):
                pts[a][pl.ds((M * i + j) * L, L)] = jnp.minimum(fa[i], fb[j])

    for jj in range(NOUT2):
        s_ref[pl.ds(jj * L, L)] = jnp.zeros((L,), jnp.float32)

    for ch in range(NCH):
        pltpu.sync_copy(rules_hbm.at[pl.ds(ch * 8 * CH, 8 * CH)], rbuf_ref)

        def _codes(i, carry, ch=ch):
            def fld(f):
                return rbuf_ref[pl.ds(f * CH + i * L, L)]
            g = ch * CH + i * L
            code_ref[pl.ds(0 * R + g, L)] = (
                fld(0) * M + (fld(1) - M)) * L
            code_ref[pl.ds(1 * R + g, L)] = (
                (fld(2) - 2 * M) * M + (fld(3) - 3 * M)) * L
            code_ref[pl.ds(2 * R + g, L)] = (
                (fld(4) - 4 * M) * M + (fld(5) - 5 * M)) * L
            obase_ref[pl.ds(0 * R + g, L)] = fld(6) * L
            obase_ref[pl.ds(1 * R + g, L)] = fld(7) * L
            return carry

        lax.fori_loop(0, CH // L, _codes, 0, unroll=False)

    def _gather_w(c0v, c1v, c2v, t):
        w0 = pt0_ref[pl.ds(c0v[t], L)]
        w1 = pt1_ref[pl.ds(c1v[t], L)]
        w2 = pt2_ref[pl.ds(c2v[t], L)]
        return jnp.minimum(jnp.minimum(w0, w1), w2)

    def _norm_block(k, accs):
        c0v = code_ref[pl.ds(0 * R + k * L, L)]
        c1v = code_ref[pl.ds(1 * R + k * L, L)]
        c2v = code_ref[pl.ds(2 * R + k * L, L)]
        accs = list(accs)
        for t in range(L):
            accs[t % 4] = accs[t % 4] + _gather_w(c0v, c1v, c2v, t)
        return tuple(accs)

    z = jnp.zeros((L,), jnp.float32)
    a0, a1, a2, a3 = lax.fori_loop(0, R // L, _norm_block, (z, z, z, z),
                                   unroll=2)
    norm = (a0 + a1) + (a2 + a3)

    inv = 1.0 / jnp.maximum(norm, 1e-12)

    for c in range(PW):
        pt0_ref[pl.ds(c * L, L)] = _bf16_round(pt0_ref[pl.ds(c * L, L)] * inv)
        pt1_ref[pl.ds(c * L, L)] = _bf16_round(pt1_ref[pl.ds(c * L, L)] * inv)
        pt2_ref[pl.ds(c * L, L)] = _bf16_round(pt2_ref[pl.ds(c * L, L)] * inv)

    def _accum_block(k, carry):
        c0v = code_ref[pl.ds(0 * R + k * L, L)]
        c1v = code_ref[pl.ds(1 * R + k * L, L)]
        c2v = code_ref[pl.ds(2 * R + k * L, L)]
        o0v = obase_ref[pl.ds(0 * R + k * L, L)]
        o1v = obase_ref[pl.ds(1 * R + k * L, L)]
        for t in range(L):
            nw = _gather_w(c0v, c1v, c2v, t)
            plsc.addupdate_scatter(s_ref, [iota + o0v[t]], nw)
            plsc.addupdate_scatter(s_ref, [iota + o1v[t]], nw)
        return carry

    lax.fori_loop(0, R // L, _accum_block, 0, unroll=2)

    acc0 = jnp.zeros((L,), jnp.float32)
    for jj in range(OUT_M):
        acc0 = acc0 + s_ref[pl.ds(jj * L, L)] * ocr0[jj]
    acc1 = jnp.zeros((L,), jnp.float32)
    for jj in range(OUT_M, NOUT2):
        scal = ocr0[jj] if jj < L else ocr1[jj - L]
        acc1 = acc1 + s_ref[pl.ds(jj * L, L)] * scal
    outb_ref[pl.ds(0, L)] = acc0
    outb_ref[pl.ds(L, L)] = acc1
    pltpu.sync_copy(outb_ref, out_hbm.at[pl.ds(wid * 2 * L, 2 * L)])


@jax.jit
def kernel(x, centers, sigmas, out_centers, input_rules, output_rules):
    xw = x.T.reshape(NVAR, NW, L).transpose(1, 0, 2).reshape(-1)
    csp = jnp.pad(jnp.concatenate([centers, sigmas], axis=0),
                  ((0, 0), (0, L - M)),
                  constant_values=1.0).reshape(-1)
    rules8 = jnp.concatenate([input_rules.T, output_rules.T],
                             axis=0).astype(jnp.int32)
    rulesf = rules8.reshape(8, NCH, CH).transpose(1, 0, 2).reshape(-1)
    ocp = jnp.pad(out_centers, (0, 32 - NOUT2))

    mesh = plsc.VectorSubcoreMesh(core_axis_name="c", subcore_axis_name="s",
                                  num_cores=NC, num_subcores=NS)
    run = functools.partial(
        pl.kernel,
        out_type=jax.ShapeDtypeStruct((NW * 2 * L,), jnp.float32),
        mesh=mesh,
        scratch_types=[
            pltpu.VMEM((NVAR * L,), jnp.float32),
            pltpu.VMEM((2 * NVAR * L,), jnp.float32),
            pltpu.VMEM((32,), jnp.float32),
            pltpu.VMEM((PW * L + L,), jnp.float32),
            pltpu.VMEM((PW * L + L,), jnp.float32),
            pltpu.VMEM((PW * L + L,), jnp.float32),
            pltpu.VMEM((8 * CH,), jnp.int32),
            pltpu.VMEM((NPAIR * R,), jnp.int32),
            pltpu.VMEM((2 * R,), jnp.int32),
            pltpu.VMEM((NOUT2 * L,), jnp.float32),
            pltpu.VMEM((2 * L,), jnp.float32),
        ],
        compiler_params=pltpu.CompilerParams(needs_layout_passes=False),
    )(_sc_body)
    flat = run(xw, csp, rulesf, ocp)
    return flat.reshape(NW, 2, L).transpose(0, 2, 1).reshape(B, 2)
